# baseline (device time: 5812 ns/iter reference)
import jax
import jax.numpy as jnp
from jax import lax
from jax.experimental import pallas as pl
from jax.experimental.pallas import tpu as pltpu

N_CHUNKS = 2


def kernel(x):
    m, n = x.shape
    blocks = m // 128
    bpc = blocks // N_CHUNKS
    rpc = m // N_CHUNKS

    def body(x_ref, out_ref, comm_ref, send_sems, recv_sems):
        my_x = lax.axis_index("x")
        my_y = lax.axis_index("y")
        nbr = (my_x, 1 - my_y)

        barrier_sem = pltpu.get_barrier_semaphore()
        pl.semaphore_signal(
            barrier_sem, inc=1, device_id=nbr,
            device_id_type=pl.DeviceIdType.MESH,
        )

        def reduce_chunk(c):
            rows = x_ref[pl.ds(c * rpc, rpc), :]
            comm_ref[0, pl.ds(c * bpc, bpc)] = jnp.sum(
                rows.reshape(bpc, 128, n), axis=2
            )

        def make_rdma(c):
            return pltpu.make_async_remote_copy(
                src_ref=comm_ref.at[0, pl.ds(c * bpc, bpc)],
                dst_ref=comm_ref.at[1, pl.ds(c * bpc, bpc)],
                send_sem=send_sems.at[c],
                recv_sem=recv_sems.at[c],
                device_id=nbr,
                device_id_type=pl.DeviceIdType.MESH,
            )

        def store_chunk(c):
            total = (
                comm_ref[0, pl.ds(c * bpc, bpc)]
                + comm_ref[1, pl.ds(c * bpc, bpc)]
            ) * (1.0 / (2 * n))
            total_t = total.T
            for i in range(bpc):
                out_ref[pl.ds((c * bpc + i) * 128, 128), :] = (
                    total_t[:, i : i + 1]
                )

        reduce_chunk(0)
        pl.semaphore_wait(barrier_sem, 1)
        rdmas = [make_rdma(c) for c in range(N_CHUNKS)]
        rdmas[0].start()
        for c in range(1, N_CHUNKS):
            reduce_chunk(c)
            rdmas[c].start()
        for c in range(N_CHUNKS):
            rdmas[c].wait_recv()
            store_chunk(c)
        for c in range(N_CHUNKS):
            rdmas[c].wait_send()

    return pl.pallas_call(
        body,
        out_shape=jax.ShapeDtypeStruct((m, 1), jnp.float32),
        in_specs=[pl.BlockSpec(memory_space=pltpu.VMEM)],
        out_specs=pl.BlockSpec(memory_space=pltpu.VMEM),
        scratch_shapes=[
            pltpu.VMEM((2, blocks, 128), jnp.float32),
            pltpu.SemaphoreType.DMA((N_CHUNKS,)),
            pltpu.SemaphoreType.DMA((N_CHUNKS,)),
        ],
        compiler_params=pltpu.CompilerParams(collective_id=0),
    )(x)


# device time: 5728 ns/iter; 1.0147x vs baseline; 1.0147x over previous
import jax
import jax.numpy as jnp
from jax import lax
from jax.experimental import pallas as pl
from jax.experimental.pallas import tpu as pltpu


def kernel(x):
    m, n = x.shape

    def body(x_ref, out_ref, comm_ref, send_sem, recv_sem):
        my_x = lax.axis_index("x")
        my_y = lax.axis_index("y")
        nbr = (my_x, 1 - my_y)

        barrier_sem = pltpu.get_barrier_semaphore()
        pl.semaphore_signal(
            barrier_sem, inc=1, device_id=nbr,
            device_id_type=pl.DeviceIdType.MESH,
        )

        comm_ref[0] = jnp.sum(x_ref[...].reshape(m // 128, 128, n), axis=2)

        pl.semaphore_wait(barrier_sem, 1)

        rdma = pltpu.make_async_remote_copy(
            src_ref=comm_ref.at[0],
            dst_ref=comm_ref.at[1],
            send_sem=send_sem,
            recv_sem=recv_sem,
            device_id=nbr,
            device_id_type=pl.DeviceIdType.MESH,
        )
        rdma.start()
        rdma.wait_recv()

        total = (comm_ref[0] + comm_ref[1]) * (1.0 / (2 * n))
        total_t = total.T
        for i in range(m // 128):
            out_ref[pl.ds(i * 128, 128), :] = total_t[:, i : i + 1]

        rdma.wait_send()

    return pl.pallas_call(
        body,
        out_shape=jax.ShapeDtypeStruct((m, 1), jnp.float32),
        in_specs=[pl.BlockSpec(memory_space=pltpu.VMEM)],
        out_specs=pl.BlockSpec(memory_space=pltpu.VMEM),
        scratch_shapes=[
            pltpu.VMEM((2, m // 128, 128), jnp.float32),
            pltpu.SemaphoreType.DMA,
            pltpu.SemaphoreType.DMA,
        ],
        compiler_params=pltpu.CompilerParams(collective_id=0),
    )(x)
